# Initial kernel scaffold; baseline (speedup 1.0000x reference)
import jax
import jax.numpy as jnp
from jax.experimental import pallas as pl


def kernel(x, W1, b1, W2, b2, Wg, bg):
    raise NotImplementedError("write your pallas kernel here")


# speedup vs baseline: 3.3747x; 3.3747x over previous
import functools
import math

import jax
import jax.numpy as jnp
from jax import lax
from jax.experimental import pallas as pl
from jax.experimental.pallas import tpu as pltpu
from jax.experimental.pallas import tpu_sc as plsc

S = 2048
D = 768
F = 3072
E = 8
K = 2
A = S * K
T = 128
NT = A // T + E
R = NT * T

NC = 2
NS = 16
NW = NC * NS
CH = A // NW
CT = S // NW
LANES = 16


def _gate_body(x_ref, wg_ref, bg_ref, ti_ref, tp_ref, aux_ref):
    x = x_ref[...]
    logits = jnp.dot(x, wg_ref[...], preferred_element_type=jnp.float32)
    logits = logits + bg_ref[...]
    iota = lax.broadcasted_iota(jnp.int32, (S, E), 1)
    m1 = jnp.max(logits, axis=1, keepdims=True)
    i1 = jnp.min(jnp.where(logits == m1, iota, E), axis=1, keepdims=True)
    rest = jnp.where(iota == i1, -jnp.inf, logits)
    m2 = jnp.max(rest, axis=1, keepdims=True)
    i2 = jnp.min(jnp.where(rest == m2, iota, E), axis=1, keepdims=True)
    p1 = 1.0 / (1.0 + jnp.exp(m2 - m1))
    ti_ref[...] = jnp.concatenate([i1, i2], axis=1)
    tp_ref[...] = jnp.concatenate([p1, 1.0 - p1], axis=1)
    z = jnp.exp(logits - m1)
    gp = z / jnp.sum(z, axis=1, keepdims=True)
    usage = jnp.mean(gp, axis=0)
    aux_ref[...] = (E * jnp.sum(usage * usage)).reshape(1, 1)


def _gate(x2d, Wg, bg):
    return pl.pallas_call(
        _gate_body,
        out_shape=(
            jax.ShapeDtypeStruct((S, K), jnp.int32),
            jax.ShapeDtypeStruct((S, K), jnp.float32),
            jax.ShapeDtypeStruct((1, 1), jnp.float32),
        ),
    )(x2d, Wg, bg.reshape(1, E))


def _routing_meta(top_idx):
    ids = top_idx.reshape(A)
    oh = (ids[:, None] == jnp.arange(E, dtype=jnp.int32)[None, :]).astype(jnp.int32)
    cum = jnp.cumsum(oh, axis=0)
    rank = jnp.sum(cum * oh, axis=1) - 1
    counts = cum[-1]
    padded = ((counts + T - 1) // T) * T
    cum_pad = jnp.cumsum(padded)
    offs = cum_pad - padded
    pos = jnp.sum(oh * offs[None, :], axis=1) + rank
    tok = jnp.arange(A, dtype=jnp.int32) // K
    n_active = (cum_pad[-1] // T).astype(jnp.int32).reshape(1)
    tile_expert = jnp.sum(
        (jnp.arange(NT, dtype=jnp.int32)[:, None] * T >= cum_pad[None, :]).astype(jnp.int32),
        axis=1)
    tile_expert = jnp.minimum(tile_expert, E - 1)
    return pos.astype(jnp.int32), tok, tile_expert, n_active


def _dispatch_body(x_hbm, tok_hbm, pos_hbm, xs_hbm, tok_v, pos_v, rows_v,
                   sem_g, sem_s):
    wid = lax.axis_index("s") * NC + lax.axis_index("c")
    base = wid * CH
    pltpu.sync_copy(tok_hbm.at[pl.ds(base, CH)], tok_v)
    pltpu.sync_copy(pos_hbm.at[pl.ds(base, CH)], pos_v)
    pltpu.async_copy(x_hbm.at[tok_v], rows_v, sem_g).wait()
    pltpu.async_copy(rows_v, xs_hbm.at[pos_v], sem_s).wait()


def _dispatch(x2d, tok, pos):
    mesh = plsc.VectorSubcoreMesh(core_axis_name="c", subcore_axis_name="s")
    return pl.kernel(
        _dispatch_body,
        out_type=jax.ShapeDtypeStruct((R, D), jnp.float32),
        mesh=mesh,
        scratch_types=[
            pltpu.VMEM((CH,), jnp.int32),
            pltpu.VMEM((CH,), jnp.int32),
            pltpu.VMEM((CH, D), jnp.float32),
            pltpu.SemaphoreType.DMA,
            pltpu.SemaphoreType.DMA,
        ],
    )(x2d, tok, pos)


def _ffn_body(te_ref, na_ref, xs_ref, w1_ref, b1_ref, w2_ref, b2_ref, y_ref):
    i = pl.program_id(0)

    @pl.when(i < na_ref[0])
    def _():
        x = xs_ref[...]
        h = jnp.dot(x, w1_ref[0], preferred_element_type=jnp.float32)
        h = h + b1_ref[0]
        h = 0.5 * h * (1.0 + lax.erf(h * (1.0 / math.sqrt(2.0))))
        y = jnp.dot(h, w2_ref[0], preferred_element_type=jnp.float32)
        y_ref[...] = y + b2_ref[0]


def _ffn(xs, W1, b1, W2, b2, tile_expert, n_active):
    grid_spec = pltpu.PrefetchScalarGridSpec(
        num_scalar_prefetch=2,
        grid=(NT,),
        in_specs=[
            pl.BlockSpec((T, D), lambda i, te, na: (i, 0)),
            pl.BlockSpec((1, D, F), lambda i, te, na: (te[i], 0, 0)),
            pl.BlockSpec((1, 1, F), lambda i, te, na: (te[i], 0, 0)),
            pl.BlockSpec((1, F, D), lambda i, te, na: (te[i], 0, 0)),
            pl.BlockSpec((1, 1, D), lambda i, te, na: (te[i], 0, 0)),
        ],
        out_specs=pl.BlockSpec((T, D), lambda i, te, na: (i, 0)),
    )
    return pl.pallas_call(
        _ffn_body,
        grid_spec=grid_spec,
        out_shape=jax.ShapeDtypeStruct((R, D), jnp.float32),
    )(tile_expert, n_active, xs, W1, b1.reshape(E, 1, F), W2,
      b2.reshape(E, 1, D))


def _combine_body(y_hbm, pos_hbm, pw_hbm, out_hbm, idx_v, w_v, rows_v, out_v,
                  sem_g):
    wid = lax.axis_index("s") * NC + lax.axis_index("c")
    for half in range(2):
        abase = wid * CH + half * (CH // 2)
        pltpu.sync_copy(pos_hbm.at[pl.ds(abase, CH // 2)], idx_v)
        pltpu.sync_copy(pw_hbm.at[pl.ds(abase, CH // 2)], w_v)
        pltpu.async_copy(y_hbm.at[idx_v], rows_v, sem_g).wait()

        def body(j, _):
            w0 = w_v[2 * j, :]
            w1 = w_v[2 * j + 1, :]
            for c in range(D // LANES):
                sl = pl.ds(c * LANES, LANES)
                out_v[j, sl] = w0 * rows_v[2 * j, sl] + w1 * rows_v[2 * j + 1, sl]
            return 0

        lax.fori_loop(0, CT // 2, body, 0)
        pltpu.sync_copy(out_v, out_hbm.at[pl.ds(wid * CT + half * (CT // 2),
                                                CT // 2)])


def _combine(y, pos, pw):
    mesh = plsc.VectorSubcoreMesh(core_axis_name="c", subcore_axis_name="s")
    return pl.kernel(
        _combine_body,
        out_type=jax.ShapeDtypeStruct((S, D), jnp.float32),
        mesh=mesh,
        scratch_types=[
            pltpu.VMEM((CH // 2,), jnp.int32),
            pltpu.VMEM((CH // 2, LANES), jnp.float32),
            pltpu.VMEM((CH // 2, D), jnp.float32),
            pltpu.VMEM((CT // 2, D), jnp.float32),
            pltpu.SemaphoreType.DMA,
        ],
    )(y, pos, pw)


def kernel(x, W1, b1, W2, b2, Wg, bg):
    x2d = x.reshape(S, D)
    top_idx, top_p, aux = _gate(x2d, Wg, bg)
    pos, tok, tile_expert, n_active = _routing_meta(top_idx)
    xs = _dispatch(x2d, tok, pos)
    y = _ffn(xs, W1, b1, W2, b2, tile_expert, n_active)
    pw = jnp.broadcast_to(top_p.reshape(A, 1), (A, LANES))
    out2d = _combine(y, pos, pw)
    return out2d.reshape(1, S, D), aux.reshape(())
